# Initial kernel scaffold; baseline (speedup 1.0000x reference)
import jax
import jax.numpy as jnp
from jax.experimental import pallas as pl


def kernel(x, edge_index, W, b):
    raise NotImplementedError("write your pallas kernel here")


# speedup vs baseline: 7.3610x; 7.3610x over previous
import functools

import jax
import jax.numpy as jnp
from jax import lax
from jax.experimental import pallas as pl
from jax.experimental.pallas import tpu as pltpu
from jax.experimental.pallas import tpu_sc as plsc

N = 10000
C = 256
K = 16
C_OUT = 256

NC = 2
NS = 16
NW = NC * NS
CPW = C // NW
CHUNK = 400
N_CHUNKS = N // CHUNK

_GDN = lax.GatherDimensionNumbers(
    offset_dims=(), collapsed_slice_dims=(0,), start_index_map=(0,))


def _dyn_gather(v, idx):
  return lax.gather(v, idx[:, None], _GDN, (1,),
                    mode=lax.GatherScatterMode.PROMISE_IN_BOUNDS)


def _sc_maxrel(xs, idx2):
  mesh = plsc.VectorSubcoreMesh(core_axis_name="c", subcore_axis_name="s")

  @functools.partial(
      pl.kernel,
      out_type=jax.ShapeDtypeStruct((C, N), jnp.float32),
      mesh=mesh,
      scratch_types=[
          pltpu.VMEM((N, CPW), jnp.float32),
          pltpu.VMEM((CHUNK, 2 * K), jnp.int32),
          pltpu.VMEM((CPW, CHUNK), jnp.float32),
      ],
      compiler_params=pltpu.CompilerParams(use_tc_tiling_on_sc=False,
                                           needs_layout_passes=False),
  )
  def body(xs_hbm, idx_hbm, m_hbm, table_v, idx_v, m_v):
    w = lax.axis_index("s") * NC + lax.axis_index("c")
    pltpu.sync_copy(xs_hbm.at[w], table_v)

    ccol = lax.iota(jnp.int32, 16)
    mask8 = ccol < 8
    hi = jnp.where(mask8, 0, 1)
    caddr = jnp.where(mask8, ccol, 15 - ccol)
    perms = [2 * p + hi for p in range(K // 2)]

    def node_body(n, _):
      jv = idx_v[n, pl.ds(0, K)]
      iv = idx_v[n, pl.ds(K, K)]
      acc = jnp.full((16,), -jnp.inf, dtype=jnp.float32)
      for p in range(K // 2):
        rj = _dyn_gather(jv, perms[p])
        ri = _dyn_gather(iv, perms[p])
        vj = plsc.load_gather(table_v, [rj, caddr])
        vi = plsc.load_gather(table_v, [ri, caddr])
        acc = jnp.maximum(acc, vj - vi)
      accf = jnp.maximum(acc, lax.rev(acc, (0,)))
      plsc.store_scatter(m_v, [ccol, jnp.full((16,), n, dtype=jnp.int32)],
                         accf, mask=mask8)
      return _

    def chunk_body(ci, _):
      base = ci * CHUNK
      pltpu.sync_copy(idx_hbm.at[pl.ds(base, CHUNK)], idx_v)
      lax.fori_loop(0, CHUNK, node_body, 0)
      pltpu.sync_copy(m_v, m_hbm.at[pl.ds(w * CPW, CPW), pl.ds(base, CHUNK)])
      return _

    lax.fori_loop(0, N_CHUNKS, chunk_body, 0)

  return body(xs, idx2)


def _tc_matmul(x_cn, m_cn, w1, w2, b2):
  def body(x_ref, m_ref, w1_ref, w2_ref, b_ref, o_ref):
    a = lax.dot_general(w1_ref[...], x_ref[...], (((1,), (0,)), ((), ())),
                        preferred_element_type=jnp.float32)
    bm = lax.dot_general(w2_ref[...], m_ref[...], (((1,), (0,)), ((), ())),
                         preferred_element_type=jnp.float32)
    o_ref[...] = jnp.maximum(a + bm + b_ref[...], 0.0)

  return pl.pallas_call(
      body,
      out_shape=jax.ShapeDtypeStruct((C_OUT, N), jnp.float32),
  )(x_cn, m_cn, w1, w2, b2)


def kernel(x, edge_index, W, b):
  x_cn = x[0, :, :, 0]
  xs = jnp.transpose(x_cn.reshape(NW, CPW, N), (0, 2, 1))
  idx2 = jnp.concatenate([edge_index[0, 0], edge_index[1, 0]], axis=-1)
  m = _sc_maxrel(xs, idx2)
  out = _tc_matmul(x_cn, m, W[:, :C], W[:, C:], b[:, None])
  return out[None, :, :, None]


# speedup vs baseline: 13.0197x; 1.7687x over previous
import functools

import jax
import jax.numpy as jnp
from jax import lax
from jax.experimental import pallas as pl
from jax.experimental.pallas import tpu as pltpu
from jax.experimental.pallas import tpu_sc as plsc

N = 10000
C = 256
K = 16
C_OUT = 256

NC = 2
NS = 16
NW = NC * NS
CPW = C // NW
CHUNK = 400
N_CHUNKS = N // CHUNK

_GDN = lax.GatherDimensionNumbers(
    offset_dims=(), collapsed_slice_dims=(0,), start_index_map=(0,))


def _dyn_gather(v, idx):
  return lax.gather(v, idx[:, None], _GDN, (1,),
                    mode=lax.GatherScatterMode.PROMISE_IN_BOUNDS)


def _sc_maxrel(x_flat, e_flat):
  mesh = plsc.VectorSubcoreMesh(core_axis_name="c", subcore_axis_name="s")

  @functools.partial(
      pl.kernel,
      out_type=jax.ShapeDtypeStruct((C, N), jnp.float32),
      mesh=mesh,
      scratch_types=[
          pltpu.VMEM((CPW * N,), jnp.float32),
          pltpu.VMEM((2, CHUNK * K), jnp.int32),
          pltpu.VMEM((2, CHUNK * K), jnp.int32),
          pltpu.VMEM((CPW, CHUNK), jnp.float32),
          pltpu.SemaphoreType.DMA,
          pltpu.SemaphoreType.DMA,
      ],
      compiler_params=pltpu.CompilerParams(use_tc_tiling_on_sc=False,
                                           needs_layout_passes=False),
  )
  def body(x_hbm, e_hbm, m_hbm, table_v, idxj_v, idxi_v, m_v, semj, semi):
    w = lax.axis_index("s") * NC + lax.axis_index("c")
    pltpu.sync_copy(x_hbm.at[pl.ds(w * (CPW * N), CPW * N)], table_v)

    ccol = lax.iota(jnp.int32, 16)
    mask8 = ccol < 8
    hi = jnp.where(mask8, 0, 1)
    caddr_n = jnp.where(mask8, ccol, 15 - ccol) * N
    perms = [2 * p + hi for p in range(K // 2)]

    def ecopy(side, base, slot):
      return pltpu.make_async_copy(
          e_hbm.at[pl.ds(side * (N * K) + base * K, CHUNK * K)],
          (idxj_v if side == 0 else idxi_v).at[slot],
          semj if side == 0 else semi)

    def process_chunk(ci, slot, prefetch):
      base = ci * CHUNK
      ecopy(0, base, slot).wait()
      ecopy(1, base, slot).wait()
      if prefetch:
        ecopy(0, base + CHUNK, 1 - slot).start()
        ecopy(1, base + CHUNK, 1 - slot).start()

      @plsc.parallel_loop(0, CHUNK, unroll=2)
      def node_body(n):
        jv = idxj_v[slot, pl.ds(n * K, K)]
        iv = idxi_v[slot, pl.ds(n * K, K)]
        acc = jnp.full((16,), -jnp.inf, dtype=jnp.float32)
        for p in range(K // 2):
          aj = _dyn_gather(jv, perms[p]) + caddr_n
          ai = _dyn_gather(iv, perms[p]) + caddr_n
          vj = plsc.load_gather(table_v, [aj])
          vi = plsc.load_gather(table_v, [ai])
          acc = jnp.maximum(acc, vj - vi)
        accf = jnp.maximum(acc, lax.rev(acc, (0,)))
        plsc.store_scatter(m_v, [ccol, jnp.full((16,), n, dtype=jnp.int32)],
                           accf, mask=mask8)

      pltpu.sync_copy(m_v, m_hbm.at[pl.ds(w * CPW, CPW), pl.ds(base, CHUNK)])

    ecopy(0, 0, 0).start()
    ecopy(1, 0, 0).start()

    def chunk_pair(cp, _):
      process_chunk(2 * cp, 0, True)
      process_chunk(2 * cp + 1, 1, True)
      return _

    lax.fori_loop(0, (N_CHUNKS - 1) // 2, chunk_pair, 0)
    process_chunk(N_CHUNKS - 1, 0, False)

  return body(x_flat, e_flat)


def _tc_matmul(x_cn, m_cn, w1, w2, b2):
  def body(x_ref, m_ref, w1_ref, w2_ref, b_ref, o_ref):
    a = lax.dot_general(w1_ref[...], x_ref[...], (((1,), (0,)), ((), ())),
                        preferred_element_type=jnp.float32)
    bm = lax.dot_general(w2_ref[...], m_ref[...], (((1,), (0,)), ((), ())),
                         preferred_element_type=jnp.float32)
    o_ref[...] = jnp.maximum(a + bm + b_ref[...], 0.0)

  return pl.pallas_call(
      body,
      out_shape=jax.ShapeDtypeStruct((C_OUT, N), jnp.float32),
  )(x_cn, m_cn, w1, w2, b2)


def kernel(x, edge_index, W, b):
  x_cn = jnp.reshape(x, (C, N))
  m = _sc_maxrel(jnp.reshape(x, (C * N,)),
                 jnp.reshape(edge_index, (2 * N * K,)))
  out = _tc_matmul(x_cn, m, W[:, :C], W[:, C:], b[:, None])
  return out[None, :, :, None]
